# baseline (speedup 1.0000x reference)
import functools

import jax
import jax.numpy as jnp
from jax import lax
from jax.experimental import pallas as pl
from jax.experimental.pallas import tpu as pltpu
from jax.experimental.pallas import tpu_sc as plsc

VOCAB = 1000000
D = 64
CHUNK = 128


def _make_gather(B: int):
    info = plsc.get_sparse_core_info()
    NC, NS = info.num_cores, info.num_subcores
    NW = NC * NS
    assert B % (NW * CHUNK) == 0
    b_per_w = B // NW
    n_chunks = b_per_w // CHUNK
    mesh = plsc.VectorSubcoreMesh(core_axis_name="c", subcore_axis_name="s")

    @functools.partial(
        pl.kernel,
        mesh=mesh,
        out_type=jax.ShapeDtypeStruct((B, D), jnp.float32),
        scratch_types=[
            pltpu.VMEM((n_chunks, CHUNK), jnp.int32),
            pltpu.VMEM((CHUNK, D), jnp.float32),
            pltpu.SemaphoreType.DMA,
        ],
        compiler_params=pltpu.CompilerParams(use_tc_tiling_on_sc=False),
    )
    def gather_kernel(x_hbm, table_hbm, out_hbm, idx_v, rows_v, sem):
        wid = lax.axis_index("s") * NC + lax.axis_index("c")
        base = wid * b_per_w
        pltpu.sync_copy(x_hbm.at[pl.ds(wid * n_chunks, n_chunks)], idx_v)

        def body(j, carry):
            pltpu.async_copy(table_hbm.at[idx_v.at[j]], rows_v, sem).wait()
            pltpu.sync_copy(rows_v, out_hbm.at[pl.ds(base + j * CHUNK, CHUNK)])
            return carry

        lax.fori_loop(0, n_chunks, body, 0, unroll=False)

    return gather_kernel


def kernel(x, table):
    B = x.shape[0] * x.shape[1]
    xf = x.reshape(B // CHUNK, CHUNK).astype(jnp.int32)
    out = _make_gather(B)(xf, table)
    return out.reshape(x.shape[0], x.shape[1], D)


# speedup vs baseline: 1.1133x; 1.1133x over previous
import functools

import jax
import jax.numpy as jnp
from jax import lax
from jax.experimental import pallas as pl
from jax.experimental.pallas import tpu as pltpu
from jax.experimental.pallas import tpu_sc as plsc

VOCAB = 1000000
D = 64
CHUNK = 128


K = 4


def _make_gather(B: int):
    info = plsc.get_sparse_core_info()
    NC, NS = info.num_cores, info.num_subcores
    NW = NC * NS
    assert B % (NW * CHUNK * K * 2) == 0
    b_per_w = B // NW
    n_chunks = b_per_w // CHUNK
    n_groups = n_chunks // K
    n_pairs = n_groups // 2
    rows_per_group = K * CHUNK
    mesh = plsc.VectorSubcoreMesh(core_axis_name="c", subcore_axis_name="s")

    @functools.partial(
        pl.kernel,
        mesh=mesh,
        out_type=jax.ShapeDtypeStruct((B, D), jnp.float32),
        scratch_types=[
            pltpu.VMEM((n_chunks, CHUNK), jnp.int32),
            pltpu.VMEM((2, rows_per_group, D), jnp.float32),
            pltpu.SemaphoreType.DMA,
            pltpu.SemaphoreType.DMA,
            pltpu.SemaphoreType.DMA,
        ],
        compiler_params=pltpu.CompilerParams(use_tc_tiling_on_sc=False),
    )
    def gather_kernel(x_hbm, table_hbm, out_hbm, idx_v, rows_v, gsem, ssem0,
                      ssem1):
        wid = lax.axis_index("s") * NC + lax.axis_index("c")
        base = wid * b_per_w
        pltpu.sync_copy(x_hbm.at[pl.ds(wid * n_chunks, n_chunks)], idx_v)

        ssems = (ssem0, ssem1)

        def run_group(g, buf, wait_store):
            store_src = rows_v.at[buf]
            store_dst = out_hbm.at[pl.ds(base + g * rows_per_group,
                                         rows_per_group)]
            if wait_store:
                pltpu.make_async_copy(store_src, store_dst, ssems[buf]).wait()
            descs = []
            for b in range(K):
                descs.append(pltpu.async_copy(
                    table_hbm.at[idx_v.at[g * K + b]],
                    rows_v.at[buf].at[pl.ds(b * CHUNK, CHUNK)], gsem))
            for d in descs:
                d.wait()
            pltpu.async_copy(store_src, store_dst, ssems[buf])

        run_group(0, 0, False)
        run_group(1, 1, False)

        def body(p, carry):
            run_group(2 * p, 0, True)
            run_group(2 * p + 1, 1, True)
            return carry

        lax.fori_loop(1, n_pairs, body, 0, unroll=False)

        last = (n_groups - 2) * rows_per_group
        pltpu.make_async_copy(
            rows_v.at[0], out_hbm.at[pl.ds(base + last, rows_per_group)],
            ssem0).wait()
        pltpu.make_async_copy(
            rows_v.at[1],
            out_hbm.at[pl.ds(base + last + rows_per_group, rows_per_group)],
            ssem1).wait()

    return gather_kernel


def kernel(x, table):
    B = x.shape[0] * x.shape[1]
    xf = x.reshape(B // CHUNK, CHUNK).astype(jnp.int32)
    out = _make_gather(B)(xf, table)
    return out.reshape(x.shape[0], x.shape[1], D)
